# baseline (speedup 1.0000x reference)
import functools
import numpy as np
import jax
import jax.numpy as jnp
from jax import lax
from jax.experimental import pallas as pl
from jax.experimental.pallas import tpu as pltpu
from jax.experimental.pallas import tpu_sc as plsc

_D_MODEL = 64
_MAX_LEN = 200
_LANES = 16
_NUM_WORKERS = 32


def _pos_encoding_np(position, d_model):
    def get_angles(pos, i, d_model):
        angle_rates = 1 / np.power(10000, 2 * (i // 2) / np.float32(d_model))
        return pos * angle_rates

    angle_rads = get_angles(np.arange(position)[:, np.newaxis],
                            np.arange(d_model)[np.newaxis, :], d_model)
    angle_rads[:, 0::2] = np.sin(angle_rads[:, 0::2])
    angle_rads[:, 1::2] = np.cos(angle_rads[:, 1::2])
    return angle_rads.astype(np.float32)


@functools.lru_cache(maxsize=None)
def _build_kernel(batch, seq, vocab):
    assert batch % _NUM_WORKERS == 0
    bpw = batch // _NUM_WORKERS
    scale = float(np.sqrt(np.float32(_D_MODEL)))
    nbuf = 2

    mesh = plsc.VectorSubcoreMesh(core_axis_name="c", subcore_axis_name="s")

    @functools.partial(
        pl.kernel,
        mesh=mesh,
        out_type=jax.ShapeDtypeStruct((batch, seq, _D_MODEL), jnp.float32),
        scratch_types=[
            pltpu.VMEM((seq, bpw), jnp.int32),
            pltpu.VMEM((seq, _D_MODEL), jnp.float32),
            [pltpu.VMEM((bpw, _D_MODEL), jnp.float32)] * nbuf,
            [pltpu.VMEM((bpw, _D_MODEL), jnp.float32)] * nbuf,
            [pltpu.SemaphoreType.DMA] * nbuf,
            [pltpu.SemaphoreType.DMA] * nbuf,
        ],
        compiler_params=pltpu.CompilerParams(use_tc_tiling_on_sc=False),
    )
    def emb(idxT_hbm, table_hbm, pos_hbm, out_hbm,
            idx_v, pos_v, gbuf, obuf, semg, semo):
        wid = lax.axis_index("s") * 2 + lax.axis_index("c")
        b0 = wid * bpw
        pltpu.sync_copy(pos_hbm, pos_v)
        pltpu.sync_copy(idxT_hbm.at[:, pl.ds(b0, bpw)], idx_v)

        def start_gather(s, b):
            return pltpu.async_copy(
                table_hbm.at[idx_v.at[s]], gbuf[b], semg[b])

        def wait_gather(s, b):
            pltpu.make_async_copy(
                table_hbm.at[idx_v.at[s]], gbuf[b], semg[b]).wait()

        def start_out(s, b):
            return pltpu.async_copy(
                obuf[b], out_hbm.at[pl.ds(b0, bpw), s], semo[b])

        def wait_out(s, b):
            pltpu.make_async_copy(
                obuf[b], out_hbm.at[pl.ds(b0, bpw), s], semo[b]).wait()

        for b in range(nbuf):
            start_gather(b, b)

        def chunk_round(k, carry):
            for b in range(nbuf):
                s = k + b
                wait_gather(s, b)

                @pl.when(s >= nbuf)
                def _():
                    wait_out(s - nbuf, b)

                for c in range(_D_MODEL // _LANES):
                    sl = pl.ds(c * _LANES, _LANES)
                    pvec = pos_v[s, sl]

                    def row_body(t, pv):
                        for u in range(4):
                            obuf[b][t * 4 + u, sl] = (
                                gbuf[b][t * 4 + u, sl] * scale + pv)
                        return pv

                    lax.fori_loop(0, bpw // 4, row_body, pvec)
                start_out(s, b)

                @pl.when(s + nbuf < seq)
                def _():
                    start_gather(s + nbuf, b)
            return carry

        lax.fori_loop(0, seq // nbuf, lambda k, c: chunk_round(k * nbuf, c), 0)
        for b in range(nbuf):
            wait_out(seq - nbuf + b, b)

    return emb


def kernel(inputs, table):
    batch, seq = inputs.shape
    vocab = table.shape[0]
    pos = jnp.asarray(_pos_encoding_np(_MAX_LEN, _D_MODEL)[:seq])
    emb = _build_kernel(batch, seq, vocab)
    return emb(inputs.T, table, pos)


# speedup vs baseline: 1.2128x; 1.2128x over previous
import functools
import numpy as np
import jax
import jax.numpy as jnp
from jax import lax
from jax.experimental import pallas as pl
from jax.experimental.pallas import tpu as pltpu
from jax.experimental.pallas import tpu_sc as plsc

_D_MODEL = 64
_MAX_LEN = 200
_LANES = 16
_NUM_WORKERS = 32


def _pos_encoding_np(position, d_model):
    def get_angles(pos, i, d_model):
        angle_rates = 1 / np.power(10000, 2 * (i // 2) / np.float32(d_model))
        return pos * angle_rates

    angle_rads = get_angles(np.arange(position)[:, np.newaxis],
                            np.arange(d_model)[np.newaxis, :], d_model)
    angle_rads[:, 0::2] = np.sin(angle_rads[:, 0::2])
    angle_rads[:, 1::2] = np.cos(angle_rads[:, 1::2])
    return angle_rads.astype(np.float32)


def _pack_body(tabT_ref, out_ref, *, scale):
    x = tabT_ref[...]
    out_ref[:, 0:_D_MODEL] = jnp.transpose(x) * scale


@functools.lru_cache(maxsize=None)
def _build_pack(vocab):
    scale = float(np.sqrt(np.float32(_D_MODEL)))
    cb = 2048
    grid = (vocab + cb - 1) // cb
    return pl.pallas_call(
        functools.partial(_pack_body, scale=scale),
        grid=(grid,),
        in_specs=[pl.BlockSpec((_D_MODEL, cb), lambda i: (0, i))],
        out_specs=pl.BlockSpec((cb, 128), lambda i: (i, 0)),
        out_shape=jax.ShapeDtypeStruct((vocab, 128), jnp.float32),
    )


@functools.lru_cache(maxsize=None)
def _build_gather(batch, seq, vocab):
    assert batch % _NUM_WORKERS == 0
    bpw = batch // _NUM_WORKERS
    nbuf = 2

    mesh = plsc.VectorSubcoreMesh(core_axis_name="c", subcore_axis_name="s")

    @functools.partial(
        pl.kernel,
        mesh=mesh,
        out_type=jax.ShapeDtypeStruct((batch, seq, _D_MODEL), jnp.float32),
        scratch_types=[
            pltpu.VMEM((seq, bpw), jnp.int32),
            pltpu.VMEM((seq, _D_MODEL), jnp.float32),
            [pltpu.VMEM((bpw, 128), jnp.float32)] * nbuf,
            [pltpu.VMEM((bpw, _D_MODEL), jnp.float32)] * nbuf,
            [pltpu.SemaphoreType.DMA] * nbuf,
            [pltpu.SemaphoreType.DMA] * nbuf,
        ],
        compiler_params=pltpu.CompilerParams(use_tc_tiling_on_sc=True),
    )
    def emb(idxT_hbm, tab2_hbm, pos_hbm, out_hbm,
            idx_v, pos_v, gbuf, obuf, semg, semo):
        wid = lax.axis_index("s") * 2 + lax.axis_index("c")
        b0 = wid * bpw
        pltpu.sync_copy(pos_hbm, pos_v)
        pltpu.sync_copy(idxT_hbm.at[:, pl.ds(b0, bpw)], idx_v)

        def prep_gather(s, b):
            pltpu.async_copy(tab2_hbm.at[idx_v.at[s]], gbuf[b], semg[b])

        def wait_gather(s, b):
            pltpu.make_async_copy(
                tab2_hbm.at[idx_v.at[s]], gbuf[b], semg[b]).wait()

        def start_out(s, b):
            return pltpu.async_copy(
                obuf[b], out_hbm.at[pl.ds(b0, bpw), s], semo[b])

        def wait_out(s, b):
            pltpu.make_async_copy(
                obuf[b], out_hbm.at[pl.ds(b0, bpw), s], semo[b]).wait()

        for b in range(nbuf):
            prep_gather(b, b)

        def chunk_round(k, carry):
            for b in range(nbuf):
                s = k + b
                wait_gather(s, b)

                @pl.when(s >= nbuf)
                def _():
                    wait_out(s - nbuf, b)

                pv = tuple(pos_v[s, pl.ds(c * _LANES, _LANES)]
                           for c in range(_D_MODEL // _LANES))

                def tok_body(t, pvs):
                    for c in range(_D_MODEL // _LANES):
                        sl = pl.ds(c * _LANES, _LANES)
                        obuf[b][t, sl] = gbuf[b][t, sl] + pvs[c]
                    return pvs

                lax.fori_loop(0, bpw, tok_body, pv)
                start_out(s, b)

                @pl.when(s + nbuf < seq)
                def _():
                    prep_gather(s + nbuf, b)
            return carry

        lax.fori_loop(0, seq // nbuf, lambda k, c: chunk_round(k * nbuf, c), 0)
        for b in range(nbuf):
            wait_out(seq - nbuf + b, b)

    return emb


def kernel(inputs, table):
    batch, seq = inputs.shape
    vocab = table.shape[0]
    pos = jnp.asarray(_pos_encoding_np(_MAX_LEN, _D_MODEL)[:seq])
    tab2 = _build_pack(vocab)(table.T)
    emb = _build_gather(batch, seq, vocab)
    return emb(inputs.T, tab2, pos)
